# Initial kernel scaffold; baseline (speedup 1.0000x reference)
import jax
import jax.numpy as jnp
from jax.experimental import pallas as pl


def kernel(x, edge_index, edge_attr, edge_attr_T, W1, b1, W2, b2):
    raise NotImplementedError("write your pallas kernel here")


# speedup vs baseline: 2.1284x; 2.1284x over previous
import functools

import jax
import jax.numpy as jnp
from jax import lax
from jax.experimental import pallas as pl
from jax.experimental.pallas import tpu as pltpu
from jax.experimental.pallas import tpu_sc as plsc

N = 10000
E = 320000
DN = 128
DE = 16
H = 32

NC = 2
NS = 16
NW = NC * NS
L = 16

PER_TILE = E // NW
CH = 400
N_CHUNKS = PER_TILE // CH
GRP = 16


def _ab_body(x_ref, w_ref, o_ref):
    o_ref[...] = jnp.dot(x_ref[...], w_ref[...],
                         preferred_element_type=jnp.float32)


def _edge_c_body(ea_ref, eat_ref, w_ref, b_ref, c1_ref, c2_ref):
    w = w_ref[...]
    b = b_ref[...]
    c1_ref[...] = jnp.dot(ea_ref[...], w, preferred_element_type=jnp.float32) + b
    c2_ref[...] = jnp.dot(eat_ref[...], w, preferred_element_type=jnp.float32) + b


def _tc_precompute(x, edge_attr, edge_attr_T, W1, b1):
    w_ab = jnp.concatenate([W1[:DN], W1[DN:2 * DN]], axis=1)
    ab = pl.pallas_call(
        _ab_body,
        grid=(10,),
        in_specs=[
            pl.BlockSpec((N // 10, DN), lambda i: (i, 0)),
            pl.BlockSpec((DN, 2 * H), lambda i: (0, 0)),
        ],
        out_specs=pl.BlockSpec((N // 10, 2 * H), lambda i: (i, 0)),
        out_shape=jax.ShapeDtypeStruct((N, 2 * H), jnp.float32),
    )(x, w_ab)

    w1c = W1[2 * DN:]
    b1r = b1.reshape(1, H)
    BE = 8000
    c1, c2 = pl.pallas_call(
        _edge_c_body,
        grid=(E // BE,),
        in_specs=[
            pl.BlockSpec((BE, DE), lambda i: (i, 0)),
            pl.BlockSpec((BE, DE), lambda i: (i, 0)),
            pl.BlockSpec((DE, H), lambda i: (0, 0)),
            pl.BlockSpec((1, H), lambda i: (0, 0)),
        ],
        out_specs=[
            pl.BlockSpec((BE, H), lambda i: (i, 0)),
            pl.BlockSpec((BE, H), lambda i: (i, 0)),
        ],
        out_shape=[
            jax.ShapeDtypeStruct((E, H), jnp.float32),
            jax.ShapeDtypeStruct((E, H), jnp.float32),
        ],
    )(edge_attr, edge_attr_T, w1c, b1r)
    return ab, c1, c2


@functools.partial(
    pl.kernel,
    out_type=jax.ShapeDtypeStruct((E,), jnp.float32),
    mesh=plsc.VectorSubcoreMesh(core_axis_name="c", subcore_axis_name="s"),
    compiler_params=pltpu.CompilerParams(needs_layout_passes=False,
                                         use_tc_tiling_on_sc=False),
    scratch_types=[
        pltpu.VMEM((CH,), jnp.int32),
        pltpu.VMEM((CH,), jnp.int32),
        pltpu.VMEM((CH, 2 * H), jnp.float32),
        pltpu.VMEM((CH, 2 * H), jnp.float32),
        pltpu.VMEM((CH * H,), jnp.float32),
        pltpu.VMEM((CH * H,), jnp.float32),
        pltpu.VMEM((3 * L,), jnp.float32),
        pltpu.VMEM((GRP * L,), jnp.float32),
        pltpu.VMEM((CH,), jnp.float32),
        pltpu.SemaphoreType.DMA,
        pltpu.SemaphoreType.DMA,
    ],
)
def _sc_edge_mlp(ab_hbm, c1_hbm, c2_hbm, src_hbm, tgt_hbm, par_hbm, out_hbm,
                 srcv, tgtv, absv, abtv, c1v, c2v, pv, stg, outv,
                 sem1, sem2):
    wid = lax.axis_index("s") * NC + lax.axis_index("c")
    base = wid * PER_TILE

    pltpu.sync_copy(par_hbm, pv)
    wd0 = pv[pl.ds(0, L)]
    wd1 = pv[pl.ds(L, L)]
    cdv = pv[pl.ds(2 * L, L)]
    col = lax.iota(jnp.int32, L) * L

    def chunk_body(k, carry):
        g = base + k * CH
        pltpu.sync_copy(src_hbm.at[pl.ds(g, CH)], srcv)
        pltpu.sync_copy(tgt_hbm.at[pl.ds(g, CH)], tgtv)
        cp1 = pltpu.async_copy(ab_hbm.at[srcv], absv, sem1)
        cp2 = pltpu.async_copy(ab_hbm.at[tgtv], abtv, sem2)
        pltpu.sync_copy(c1_hbm.at[pl.ds(g * H, CH * H)], c1v)
        pltpu.sync_copy(c2_hbm.at[pl.ds(g * H, CH * H)], c2v)
        cp1.wait()
        cp2.wait()

        def grp_body(blk, carry2):
            e0 = blk * GRP
            for j in range(GRP):
                e = e0 + j
                s0 = absv[e, pl.ds(0, L)]
                s1 = absv[e, pl.ds(L, L)]
                s2 = absv[e, pl.ds(2 * L, L)]
                s3 = absv[e, pl.ds(3 * L, L)]
                t0 = abtv[e, pl.ds(0, L)]
                t1 = abtv[e, pl.ds(L, L)]
                t2 = abtv[e, pl.ds(2 * L, L)]
                t3 = abtv[e, pl.ds(3 * L, L)]
                c10 = c1v[pl.ds(e * H, L)]
                c11 = c1v[pl.ds(e * H + L, L)]
                c20 = c2v[pl.ds(e * H, L)]
                c21 = c2v[pl.ds(e * H + L, L)]
                zero = jnp.zeros((L,), jnp.float32)
                h1a = jnp.maximum(s0 + t2 + c10, zero)
                h1b = jnp.maximum(s1 + t3 + c11, zero)
                h2a = jnp.maximum(t0 + s2 + c20, zero)
                h2b = jnp.maximum(t1 + s3 + c21, zero)
                t = (h1a + h2a) * wd0 + (h1b + h2b) * wd1
                stg[pl.ds(j * L, L)] = t
            sums = plsc.load_gather(stg, [col])
            for kk in range(1, L):
                sums = sums + plsc.load_gather(stg, [col + kk])
            z = sums + cdv
            outv[pl.ds(e0, L)] = 1.0 / (1.0 + jnp.exp(-z))
            return carry2

        lax.fori_loop(0, CH // GRP, grp_body, 0, unroll=False)
        pltpu.sync_copy(outv, out_hbm.at[pl.ds(g, CH)])
        return carry

    lax.fori_loop(0, N_CHUNKS, chunk_body, 0, unroll=False)


def kernel(x, edge_index, edge_attr, edge_attr_T, W1, b1, W2, b2):
    ab, c1, c2 = _tc_precompute(x, edge_attr, edge_attr_T, W1, b1)
    wd = 0.5 * (W2[:, 1] - W2[:, 0])
    cd = b2[1] - b2[0]
    params = jnp.concatenate([wd, jnp.full((L,), cd, jnp.float32)])
    src = edge_index[0].astype(jnp.int32)
    tgt = edge_index[1].astype(jnp.int32)
    out = _sc_edge_mlp(ab, c1.reshape(E * H), c2.reshape(E * H),
                       src, tgt, params)
    return out.reshape(E, 1)
